# baseline (speedup 1.0000x reference)
import functools

import jax
import jax.numpy as jnp
from jax import lax
from jax.experimental import pallas as pl
from jax.experimental.pallas import tpu as pltpu
from jax.experimental.pallas import tpu_sc as plsc

B, L, V, D, O = 16384, 200, 10, 8, 4
NC, NS, LANES = 2, 16, 16
NW = NC * NS
RW = B // NW
NB = 4
NP = 128
CR = 128
NCH = RW // CR


def _rsqrt(x):
    i = plsc.bitcast(x, jnp.int32)
    i = 0x5F3759DF - lax.shift_right_arithmetic(i, 1)
    y = plsc.bitcast(i, jnp.float32)
    for _ in range(3):
        y = y * (1.5 - 0.5 * x * y * y)
    return y


def _body(x_hbm, params_hbm, out_hbm, xbufa, xbufb, outbuf, counts, pbuf,
          sema, semb):
    xbufs = (xbufa, xbufb)
    sems = (sema, semb)
    wid = lax.axis_index("s") * NC + lax.axis_index("c")
    base = wid * RW

    pltpu.sync_copy(params_hbm, pbuf)

    pvecs = [pbuf[pl.ds(i * LANES, LANES)] for i in range(NP // LANES)]

    def scal(i):
        return pvecs[i // LANES][i % LANES]

    t = [[scal(v * D + d) for d in range(D)] for v in range(V)]
    w = [[scal(V * D + d * O + j) for j in range(O)] for d in range(D)]
    m = [[sum(t[v][d] * w[d][j] for d in range(D)) * (1.0 / L)
          for j in range(O)] for v in range(V)]
    off = V * D + D * O
    bs = [scal(off + j) for j in range(O)]
    gs = [scal(off + O + j) for j in range(O)]
    zs = [scal(off + 2 * O + j) for j in range(O)]

    lane = lax.iota(jnp.int32, LANES)
    ones = jnp.ones((LANES,), jnp.float32)
    zeros = jnp.zeros((LANES,), jnp.float32)
    lane_bank = [lane + nb * V * LANES for nb in range(NB)]
    cols = [jnp.full((LANES,), k, dtype=jnp.int32) for k in range(L)]

    def make_group_body(c, xbuf):
        def group_body(g, carry):
            for v in range(NB * V):
                counts[pl.ds(v * LANES, LANES)] = zeros
            rows = g * LANES + lane

            for k in range(L):
                xv = plsc.load_gather(xbuf, [rows, cols[k]])
                plsc.addupdate_scatter(
                    counts, [xv * LANES + lane_bank[k % NB]], ones)

            cvs = [counts[pl.ds(v * LANES, LANES)]
                   + counts[pl.ds((V + v) * LANES, LANES)]
                   + counts[pl.ds((2 * V + v) * LANES, LANES)]
                   + counts[pl.ds((3 * V + v) * LANES, LANES)]
                   for v in range(V)]
            h = []
            for j in range(O):
                acc = cvs[0] * m[0][j]
                for v in range(1, V):
                    acc = acc + cvs[v] * m[v][j]
                h.append(acc + bs[j])
            mu = (h[0] + h[1] + h[2] + h[3]) * 0.25
            d = [hj - mu for hj in h]
            var = (d[0] * d[0] + d[1] * d[1] + d[2] * d[2]
                   + d[3] * d[3]) * 0.25
            r = _rsqrt(var + 1e-5)
            orow = (c * CR + g * LANES) + lane
            for j in range(O):
                o = d[j] * (r * gs[j]) + zs[j]
                plsc.store_scatter(outbuf, [orow * O + j], o)
            return carry
        return group_body

    copies = [
        pltpu.async_copy(x_hbm.at[pl.ds(base + c * CR, CR)],
                         xbufs[c % 2], sems[c % 2])
        for c in range(2)
    ]
    for c in range(NCH):
        copies[c].wait()
        if c + 2 < NCH:
            copies.append(
                pltpu.async_copy(x_hbm.at[pl.ds(base + (c + 2) * CR, CR)],
                                 xbufs[c % 2], sems[c % 2]))
        lax.fori_loop(0, CR // LANES, make_group_body(c, xbufs[c % 2]), 0)

    pltpu.sync_copy(outbuf, out_hbm.at[pl.ds(base * O, RW * O)])


_sc_call = functools.partial(
    pl.kernel,
    out_type=jax.ShapeDtypeStruct((B * O,), jnp.float32),
    mesh=plsc.VectorSubcoreMesh(core_axis_name="c", subcore_axis_name="s"),
    scratch_types=[
        pltpu.VMEM((CR, L), jnp.int32),
        pltpu.VMEM((CR, L), jnp.int32),
        pltpu.VMEM((RW * O,), jnp.float32),
        pltpu.VMEM((NB * V * LANES,), jnp.float32),
        pltpu.VMEM((NP,), jnp.float32),
        pltpu.SemaphoreType.DMA,
        pltpu.SemaphoreType.DMA,
    ],
    compiler_params=pltpu.CompilerParams(
        use_tc_tiling_on_sc=True, needs_layout_passes=False),
)(_body)


def kernel(x, table, W, b, gamma, beta):
    params = jnp.concatenate([
        table.ravel(), W.ravel(), b, gamma, beta,
        jnp.zeros((NP - (V * D + D * O + 3 * O),), jnp.float32),
    ])
    return _sc_call(x, params).reshape(B, O)


# speedup vs baseline: 1.0018x; 1.0018x over previous
import functools

import jax
import jax.numpy as jnp
from jax import lax
from jax.experimental import pallas as pl
from jax.experimental.pallas import tpu as pltpu
from jax.experimental.pallas import tpu_sc as plsc

B, L, V, D, O = 16384, 200, 10, 8, 4
NC, NS, LANES = 2, 16, 16
NW = NC * NS
RW = B // NW
NB = 4
NP = 128
CR = 128
NCH = RW // CR


def _rsqrt(x):
    i = plsc.bitcast(x, jnp.int32)
    i = 0x5F3759DF - lax.shift_right_arithmetic(i, 1)
    y = plsc.bitcast(i, jnp.float32)
    for _ in range(3):
        y = y * (1.5 - 0.5 * x * y * y)
    return y


def _body(x_hbm, params_hbm, out_hbm, xbufa, xbufb, outbuf, counts, pbuf,
          sema, semb):
    xbufs = (xbufa, xbufb)
    sems = (sema, semb)
    wid = lax.axis_index("s") * NC + lax.axis_index("c")
    base = wid * RW

    pltpu.sync_copy(params_hbm, pbuf)

    pvecs = [pbuf[pl.ds(i * LANES, LANES)] for i in range(NP // LANES)]

    def scal(i):
        return pvecs[i // LANES][i % LANES]

    t = [[scal(v * D + d) for d in range(D)] for v in range(V)]
    w = [[scal(V * D + d * O + j) for j in range(O)] for d in range(D)]
    m = [[sum(t[v][d] * w[d][j] for d in range(D)) * (1.0 / L)
          for j in range(O)] for v in range(V)]
    off = V * D + D * O
    bs = [scal(off + j) for j in range(O)]
    gs = [scal(off + O + j) for j in range(O)]
    zs = [scal(off + 2 * O + j) for j in range(O)]

    lane = lax.iota(jnp.int32, LANES)
    ones = jnp.ones((LANES,), jnp.float32)
    zeros = jnp.zeros((LANES,), jnp.float32)
    lane_bank = [lane + nb * V * LANES for nb in range(NB)]
    cols = [jnp.full((LANES,), k, dtype=jnp.int32) for k in range(L)]

    def make_group_body(c, xbuf):
        def group_body(g, carry):
            for v in range(NB * V):
                counts[pl.ds(v * LANES, LANES)] = zeros
            rows = g * LANES + lane

            for k in range(L):
                xv = plsc.load_gather(xbuf, [rows, cols[k]])
                plsc.addupdate_scatter(
                    counts, [xv * LANES + lane_bank[k % NB]], ones)

            cvs = [counts[pl.ds(v * LANES, LANES)]
                   + counts[pl.ds((V + v) * LANES, LANES)]
                   + counts[pl.ds((2 * V + v) * LANES, LANES)]
                   + counts[pl.ds((3 * V + v) * LANES, LANES)]
                   for v in range(V)]
            h = []
            for j in range(O):
                acc = cvs[0] * m[0][j]
                for v in range(1, V):
                    acc = acc + cvs[v] * m[v][j]
                h.append(acc + bs[j])
            mu = (h[0] + h[1] + h[2] + h[3]) * 0.25
            d = [hj - mu for hj in h]
            var = (d[0] * d[0] + d[1] * d[1] + d[2] * d[2]
                   + d[3] * d[3]) * 0.25
            r = _rsqrt(var + 1e-5)
            orow = (c * CR + g * LANES) + lane
            for j in range(O):
                o = d[j] * (r * gs[j]) + zs[j]
                plsc.store_scatter(outbuf, [orow * O + j], o)
            return carry
        return group_body

    copies = [
        pltpu.async_copy(x_hbm.at[pl.ds(base + c * CR, CR)],
                         xbufs[c % 2], sems[c % 2])
        for c in range(2)
    ]
    for c in range(NCH):
        copies[c].wait()
        lax.fori_loop(0, CR // LANES, make_group_body(c, xbufs[c % 2]), 0)
        if c + 2 < NCH:
            copies.append(
                pltpu.async_copy(x_hbm.at[pl.ds(base + (c + 2) * CR, CR)],
                                 xbufs[c % 2], sems[c % 2]))

    pltpu.sync_copy(outbuf, out_hbm.at[pl.ds(base * O, RW * O)])


_sc_call = functools.partial(
    pl.kernel,
    out_type=jax.ShapeDtypeStruct((B * O,), jnp.float32),
    mesh=plsc.VectorSubcoreMesh(core_axis_name="c", subcore_axis_name="s"),
    scratch_types=[
        pltpu.VMEM((CR, L), jnp.int32),
        pltpu.VMEM((CR, L), jnp.int32),
        pltpu.VMEM((RW * O,), jnp.float32),
        pltpu.VMEM((NB * V * LANES,), jnp.float32),
        pltpu.VMEM((NP,), jnp.float32),
        pltpu.SemaphoreType.DMA,
        pltpu.SemaphoreType.DMA,
    ],
    compiler_params=pltpu.CompilerParams(
        use_tc_tiling_on_sc=True, needs_layout_passes=False),
)(_body)


def kernel(x, table, W, b, gamma, beta):
    params = jnp.concatenate([
        table.ravel(), W.ravel(), b, gamma, beta,
        jnp.zeros((NP - (V * D + D * O + 3 * O),), jnp.float32),
    ])
    return _sc_call(x, params).reshape(B, O)


# speedup vs baseline: 1.5528x; 1.5500x over previous
import functools

import jax
import jax.numpy as jnp
from jax import lax
from jax.experimental import pallas as pl
from jax.experimental.pallas import tpu as pltpu
from jax.experimental.pallas import tpu_sc as plsc

B, L, V, D, O = 16384, 200, 10, 8, 4
NC, NS, LANES = 2, 16, 16
NW = NC * NS
RW = B // NW
NB = 4
NP = 128
CR = 128
NCH = RW // CR


def _rsqrt(x):
    i = plsc.bitcast(x, jnp.int32)
    i = 0x5F3759DF - lax.shift_right_arithmetic(i, 1)
    y = plsc.bitcast(i, jnp.float32)
    for _ in range(3):
        y = y * (1.5 - 0.5 * x * y * y)
    return y


def _body(x_hbm, params_hbm, out_hbm, xbufa, xbufb, outbuf, counts, pbuf,
          sema, semb):
    xbufs = (xbufa, xbufb)
    sems = (sema, semb)
    wid = lax.axis_index("s") * NC + lax.axis_index("c")
    base = wid * RW

    pltpu.sync_copy(params_hbm, pbuf)

    pvecs = [pbuf[pl.ds(i * LANES, LANES)] for i in range(NP // LANES)]

    def scal(i):
        return pvecs[i // LANES][i % LANES]

    t = [[scal(v * D + d) for d in range(D)] for v in range(V)]
    w = [[scal(V * D + d * O + j) for j in range(O)] for d in range(D)]
    m = [[sum(t[v][d] * w[d][j] for d in range(D)) * (1.0 / L)
          for j in range(O)] for v in range(V)]
    off = V * D + D * O
    bs = [scal(off + j) for j in range(O)]
    gs = [scal(off + O + j) for j in range(O)]
    zs = [scal(off + 2 * O + j) for j in range(O)]

    lane = lax.iota(jnp.int32, LANES)
    ones = jnp.ones((LANES,), jnp.float32)
    zeros = jnp.zeros((LANES,), jnp.float32)
    lane_bank = [lane + nb * V * LANES for nb in range(NB)]

    def make_group_body(c, xbuf):
        def group_body(g, carry):
            for v in range(NB * V):
                counts[pl.ds(v * LANES, LANES)] = zeros
            rows = g * LANES + lane

            @plsc.parallel_loop(0, L, step=NB, unroll=5)
            def l_body(l):
                for nb in range(NB):
                    col = jnp.full((LANES,), l + nb, dtype=jnp.int32)
                    xv = plsc.load_gather(xbuf, [rows, col])
                    plsc.addupdate_scatter(
                        counts, [xv * LANES + lane_bank[nb]], ones)

            cvs = [counts[pl.ds(v * LANES, LANES)]
                   + counts[pl.ds((V + v) * LANES, LANES)]
                   + counts[pl.ds((2 * V + v) * LANES, LANES)]
                   + counts[pl.ds((3 * V + v) * LANES, LANES)]
                   for v in range(V)]
            h = []
            for j in range(O):
                acc = cvs[0] * m[0][j]
                for v in range(1, V):
                    acc = acc + cvs[v] * m[v][j]
                h.append(acc + bs[j])
            mu = (h[0] + h[1] + h[2] + h[3]) * 0.25
            d = [hj - mu for hj in h]
            var = (d[0] * d[0] + d[1] * d[1] + d[2] * d[2]
                   + d[3] * d[3]) * 0.25
            r = _rsqrt(var + 1e-5)
            orow = (c * CR + g * LANES) + lane
            for j in range(O):
                o = d[j] * (r * gs[j]) + zs[j]
                plsc.store_scatter(outbuf, [orow * O + j], o)
            return carry
        return group_body

    copies = [
        pltpu.async_copy(x_hbm.at[pl.ds(base + c * CR, CR)],
                         xbufs[c % 2], sems[c % 2])
        for c in range(2)
    ]
    for c in range(NCH):
        copies[c].wait()
        lax.fori_loop(0, CR // LANES, make_group_body(c, xbufs[c % 2]), 0)
        if c + 2 < NCH:
            copies.append(
                pltpu.async_copy(x_hbm.at[pl.ds(base + (c + 2) * CR, CR)],
                                 xbufs[c % 2], sems[c % 2]))

    pltpu.sync_copy(outbuf, out_hbm.at[pl.ds(base * O, RW * O)])


_sc_call = functools.partial(
    pl.kernel,
    out_type=jax.ShapeDtypeStruct((B * O,), jnp.float32),
    mesh=plsc.VectorSubcoreMesh(core_axis_name="c", subcore_axis_name="s"),
    scratch_types=[
        pltpu.VMEM((CR, L), jnp.int32),
        pltpu.VMEM((CR, L), jnp.int32),
        pltpu.VMEM((RW * O,), jnp.float32),
        pltpu.VMEM((NB * V * LANES,), jnp.float32),
        pltpu.VMEM((NP,), jnp.float32),
        pltpu.SemaphoreType.DMA,
        pltpu.SemaphoreType.DMA,
    ],
    compiler_params=pltpu.CompilerParams(
        use_tc_tiling_on_sc=True, needs_layout_passes=False),
)(_body)


def kernel(x, table, W, b, gamma, beta):
    params = jnp.concatenate([
        table.ravel(), W.ravel(), b, gamma, beta,
        jnp.zeros((NP - (V * D + D * O + 3 * O),), jnp.float32),
    ])
    return _sc_call(x, params).reshape(B, O)


# speedup vs baseline: 1.7217x; 1.1088x over previous
import functools

import jax
import jax.numpy as jnp
from jax import lax
from jax.experimental import pallas as pl
from jax.experimental.pallas import tpu as pltpu
from jax.experimental.pallas import tpu_sc as plsc

B, L, V, D, O = 16384, 200, 10, 8, 4
NC, NS, LANES = 2, 16, 16
NW = NC * NS
RW = B // NW
NB = 4
NP = 128
CR = 128
NCH = RW // CR


def _rsqrt(x):
    i = plsc.bitcast(x, jnp.int32)
    i = 0x5F3759DF - lax.shift_right_arithmetic(i, 1)
    y = plsc.bitcast(i, jnp.float32)
    for _ in range(3):
        y = y * (1.5 - 0.5 * x * y * y)
    return y


def _body(x_hbm, params_hbm, out_hbm, xbufa, xbufb, outbuf, counts, pbuf,
          sema, semb):
    xbufs = (xbufa, xbufb)
    sems = (sema, semb)
    wid = lax.axis_index("s") * NC + lax.axis_index("c")
    base = wid * RW

    pltpu.sync_copy(params_hbm, pbuf)

    pvecs = [pbuf[pl.ds(i * LANES, LANES)] for i in range(NP // LANES)]

    def scal(i):
        return pvecs[i // LANES][i % LANES]

    t = [[scal(v * D + d) for d in range(D)] for v in range(V)]
    w = [[scal(V * D + d * O + j) for j in range(O)] for d in range(D)]
    m = [[sum(t[v][d] * w[d][j] for d in range(D)) * (1.0 / L)
          for j in range(O)] for v in range(V)]
    off = V * D + D * O
    bs = [scal(off + j) for j in range(O)]
    gs = [scal(off + O + j) for j in range(O)]
    zs = [scal(off + 2 * O + j) for j in range(O)]

    lane = lax.iota(jnp.int32, LANES)
    ones = jnp.ones((LANES,), jnp.float32)
    zeros = jnp.zeros((LANES,), jnp.float32)
    lane_bank = [lane + nb * V * LANES for nb in range(NB)]

    def make_group_body(c, xbuf):
        def group_body(g, carry):
            for v in range(NB * V):
                counts[pl.ds(v * LANES, LANES)] = zeros
            rows = g * LANES + lane
            rowoff = rows * L

            @plsc.parallel_loop(0, L, step=NB, unroll=5)
            def l_body(l):
                xbase = rowoff + l
                for nb in range(NB):
                    xv = plsc.load_gather(xbuf, [xbase + nb])
                    plsc.addupdate_scatter(
                        counts, [xv * LANES + lane_bank[nb]], ones)

            cvs = [counts[pl.ds(v * LANES, LANES)]
                   + counts[pl.ds((V + v) * LANES, LANES)]
                   + counts[pl.ds((2 * V + v) * LANES, LANES)]
                   + counts[pl.ds((3 * V + v) * LANES, LANES)]
                   for v in range(V)]
            h = []
            for j in range(O):
                acc = cvs[0] * m[0][j]
                for v in range(1, V):
                    acc = acc + cvs[v] * m[v][j]
                h.append(acc + bs[j])
            mu = (h[0] + h[1] + h[2] + h[3]) * 0.25
            d = [hj - mu for hj in h]
            var = (d[0] * d[0] + d[1] * d[1] + d[2] * d[2]
                   + d[3] * d[3]) * 0.25
            r = _rsqrt(var + 1e-5)
            orow = (c * CR + g * LANES) + lane
            for j in range(O):
                o = d[j] * (r * gs[j]) + zs[j]
                plsc.store_scatter(outbuf, [orow * O + j], o)
            return carry
        return group_body

    copies = [
        pltpu.async_copy(x_hbm.at[pl.ds((base + c * CR) * L, CR * L)],
                         xbufs[c % 2], sems[c % 2])
        for c in range(2)
    ]
    for c in range(NCH):
        copies[c].wait()
        lax.fori_loop(0, CR // LANES, make_group_body(c, xbufs[c % 2]), 0)
        if c + 2 < NCH:
            copies.append(
                pltpu.async_copy(
                    x_hbm.at[pl.ds((base + (c + 2) * CR) * L, CR * L)],
                    xbufs[c % 2], sems[c % 2]))

    pltpu.sync_copy(outbuf, out_hbm.at[pl.ds(base * O, RW * O)])


_sc_call = functools.partial(
    pl.kernel,
    out_type=jax.ShapeDtypeStruct((B * O,), jnp.float32),
    mesh=plsc.VectorSubcoreMesh(core_axis_name="c", subcore_axis_name="s"),
    scratch_types=[
        pltpu.VMEM((CR * L,), jnp.int32),
        pltpu.VMEM((CR * L,), jnp.int32),
        pltpu.VMEM((RW * O,), jnp.float32),
        pltpu.VMEM((NB * V * LANES,), jnp.float32),
        pltpu.VMEM((NP,), jnp.float32),
        pltpu.SemaphoreType.DMA,
        pltpu.SemaphoreType.DMA,
    ],
    compiler_params=pltpu.CompilerParams(
        use_tc_tiling_on_sc=False, needs_layout_passes=False),
)(_body)


def kernel(x, table, W, b, gamma, beta):
    params = jnp.concatenate([
        table.ravel(), W.ravel(), b, gamma, beta,
        jnp.zeros((NP - (V * D + D * O + 3 * O),), jnp.float32),
    ])
    return _sc_call(x.reshape(-1), params).reshape(B, O)
